# initial kernel scaffold (unmeasured)
import jax
import jax.numpy as jnp
from jax import lax
from jax.experimental import pallas as pl
from jax.experimental.pallas import tpu as pltpu


def kernel(
    x,
):
    def body(*refs):
        pass

    out_shape = jax.ShapeDtypeStruct(..., jnp.float32)
    return pl.pallas_call(body, out_shape=out_shape)(...)


# baseline (device time: 587803 ns/iter reference)
import jax
import jax.numpy as jnp
from jax import lax
from jax.experimental import pallas as pl
from jax.experimental.pallas import tpu as pltpu

Z = 4


def kernel(x):
    _, m, n_total = x.shape
    n_per = n_total // Z

    def body(
        x_hbm,
        out_hbm,
        r0,
        r1,
        t,
        copy_sem,
        out_sem,
        send_sems,
        recv_sems,
        credit_sem,
    ):
        my_x = lax.axis_index("x")
        my_y = lax.axis_index("y")
        my_z = lax.axis_index("z")
        right = (my_z + 1) % Z
        left = (my_z - 1) % Z

        def chunk(c):
            return x_hbm.at[0, :, pl.ds(c * n_per, n_per)]

        def rdma(src, dst, h, target):
            return pltpu.make_async_remote_copy(
                src_ref=src,
                dst_ref=dst,
                send_sem=send_sems.at[h],
                recv_sem=recv_sems.at[h],
                device_id=(my_x, my_y, target),
                device_id_type=pl.DeviceIdType.MESH,
            )

        barrier = pltpu.get_barrier_semaphore()
        for nbr in (left, right):
            pl.semaphore_signal(
                barrier,
                inc=1,
                device_id=(my_x, my_y, nbr),
                device_id_type=pl.DeviceIdType.MESH,
            )
        pl.semaphore_wait(barrier, 2)

        send0 = rdma(chunk((my_z - 1) % Z), r0, 0, right)
        send0.start()
        tload = pltpu.make_async_copy(chunk((my_z - 2) % Z), t, copy_sem)
        tload.start()
        recv0 = rdma(chunk(0), r0, 0, my_z)
        recv0.wait_recv()
        tload.wait()
        r0[:, :] = r0[:, :] + t[:, :]

        send1 = rdma(r0, r1, 1, right)
        send1.start()
        recv1 = rdma(r0, r1, 1, my_z)
        recv1.wait_recv()
        tload = pltpu.make_async_copy(chunk((my_z - 3) % Z), t, copy_sem)
        tload.start()
        tload.wait()
        r1[:, :] = r1[:, :] + t[:, :]

        send1.wait_send()
        pl.semaphore_signal(
            credit_sem,
            inc=1,
            device_id=(my_x, my_y, left),
            device_id_type=pl.DeviceIdType.MESH,
        )
        pl.semaphore_wait(credit_sem, 1)

        send2 = rdma(r1, r0, 2, right)
        send2.start()
        recv2 = rdma(r1, r0, 2, my_z)
        recv2.wait_recv()
        tload = pltpu.make_async_copy(chunk(my_z), t, copy_sem)
        tload.start()
        tload.wait()
        r0[:, :] = r0[:, :] + t[:, :]

        outcopy = pltpu.make_async_copy(r0, out_hbm, out_sem)
        outcopy.start()
        outcopy.wait()

        send0.wait_send()
        send2.wait_send()

    return pl.pallas_call(
        body,
        out_shape=jax.ShapeDtypeStruct((m, n_per), x.dtype),
        in_specs=[pl.BlockSpec(memory_space=pl.ANY)],
        out_specs=pl.BlockSpec(memory_space=pl.ANY),
        scratch_shapes=[
            pltpu.VMEM((m, n_per), x.dtype),
            pltpu.VMEM((m, n_per), x.dtype),
            pltpu.VMEM((m, n_per), x.dtype),
            pltpu.SemaphoreType.DMA,
            pltpu.SemaphoreType.DMA,
            pltpu.SemaphoreType.DMA((Z - 1,)),
            pltpu.SemaphoreType.DMA((Z - 1,)),
            pltpu.SemaphoreType.REGULAR,
        ],
        compiler_params=pltpu.CompilerParams(
            collective_id=0,
            vmem_limit_bytes=60 * 1024 * 1024,
        ),
    )(x)


# device time: 576129 ns/iter; 1.0203x vs baseline; 1.0203x over previous
import jax
import jax.numpy as jnp
from jax import lax
from jax.experimental import pallas as pl
from jax.experimental.pallas import tpu as pltpu

Z = 4


def kernel(x):
    _, m, n_total = x.shape
    n_per = n_total // Z

    def body(
        x_hbm,
        out_hbm,
        r0,
        r1,
        t,
        copy_sem,
        out_sem,
        send_sems,
        recv_sems,
        credit_sem,
    ):
        my_x = lax.axis_index("x")
        my_y = lax.axis_index("y")
        my_z = lax.axis_index("z")
        right = (my_z + 1) % Z
        left = (my_z - 1) % Z

        def chunk(c):
            return x_hbm.at[0, :, pl.ds(c * n_per, n_per)]

        def rdma(src, dst, h, target):
            return pltpu.make_async_remote_copy(
                src_ref=src,
                dst_ref=dst,
                send_sem=send_sems.at[h],
                recv_sem=recv_sems.at[h],
                device_id=(my_x, my_y, target),
                device_id_type=pl.DeviceIdType.MESH,
            )

        barrier = pltpu.get_barrier_semaphore()
        for nbr in (left, right):
            pl.semaphore_signal(
                barrier,
                inc=1,
                device_id=(my_x, my_y, nbr),
                device_id_type=pl.DeviceIdType.MESH,
            )
        pl.semaphore_wait(barrier, 2)

        send0 = rdma(chunk((my_z - 1) % Z), r0, 0, right)
        send0.start()
        tload = pltpu.make_async_copy(chunk((my_z - 2) % Z), t, copy_sem)
        tload.start()
        recv0 = rdma(chunk(0), r0, 0, my_z)
        recv0.wait_recv()
        tload.wait()
        r0[:, :] = r0[:, :] + t[:, :]

        send1 = rdma(r0, r1, 1, right)
        send1.start()
        tload = pltpu.make_async_copy(chunk((my_z - 3) % Z), t, copy_sem)
        tload.start()
        recv1 = rdma(r0, r1, 1, my_z)
        recv1.wait_recv()
        tload.wait()
        r1[:, :] = r1[:, :] + t[:, :]

        send1.wait_send()
        pl.semaphore_signal(
            credit_sem,
            inc=1,
            device_id=(my_x, my_y, left),
            device_id_type=pl.DeviceIdType.MESH,
        )
        pl.semaphore_wait(credit_sem, 1)

        send2 = rdma(r1, r0, 2, right)
        send2.start()
        tload = pltpu.make_async_copy(chunk(my_z), t, copy_sem)
        tload.start()
        recv2 = rdma(r1, r0, 2, my_z)
        recv2.wait_recv()
        tload.wait()
        r0[:, :] = r0[:, :] + t[:, :]

        outcopy = pltpu.make_async_copy(r0, out_hbm, out_sem)
        outcopy.start()
        outcopy.wait()

        send0.wait_send()
        send2.wait_send()

    return pl.pallas_call(
        body,
        out_shape=jax.ShapeDtypeStruct((m, n_per), x.dtype),
        in_specs=[pl.BlockSpec(memory_space=pl.ANY)],
        out_specs=pl.BlockSpec(memory_space=pl.ANY),
        scratch_shapes=[
            pltpu.VMEM((m, n_per), x.dtype),
            pltpu.VMEM((m, n_per), x.dtype),
            pltpu.VMEM((m, n_per), x.dtype),
            pltpu.SemaphoreType.DMA,
            pltpu.SemaphoreType.DMA,
            pltpu.SemaphoreType.DMA((Z - 1,)),
            pltpu.SemaphoreType.DMA((Z - 1,)),
            pltpu.SemaphoreType.REGULAR,
        ],
        compiler_params=pltpu.CompilerParams(
            collective_id=0,
            vmem_limit_bytes=60 * 1024 * 1024,
        ),
    )(x)


# device time: 562960 ns/iter; 1.0441x vs baseline; 1.0234x over previous
import jax
import jax.numpy as jnp
from jax import lax
from jax.experimental import pallas as pl
from jax.experimental.pallas import tpu as pltpu

Z = 4
H = 2


def kernel(x):
    _, m, n_total = x.shape
    n_per = n_total // Z
    n_half = n_per // H

    def body(
        x_hbm,
        out_hbm,
        r0,
        r1,
        t,
        copy_sems,
        out_sems,
        send_sems,
        recv_sems,
        credit_sem,
    ):
        my_x = lax.axis_index("x")
        my_y = lax.axis_index("y")
        my_z = lax.axis_index("z")
        right = (my_z + 1) % Z
        left = (my_z - 1) % Z

        def chunk(c, h):
            return x_hbm.at[0, :, pl.ds(c * n_per + h * n_half, n_half)]

        def half(ref, h):
            return ref.at[:, pl.ds(h * n_half, n_half)]

        def rdma(src, dst, hop, h, target):
            return pltpu.make_async_remote_copy(
                src_ref=src,
                dst_ref=dst,
                send_sem=send_sems.at[hop, h],
                recv_sem=recv_sems.at[hop, h],
                device_id=(my_x, my_y, target),
                device_id_type=pl.DeviceIdType.MESH,
            )

        barrier = pltpu.get_barrier_semaphore()
        for nbr in (left, right):
            pl.semaphore_signal(
                barrier,
                inc=1,
                device_id=(my_x, my_y, nbr),
                device_id_type=pl.DeviceIdType.MESH,
            )
        pl.semaphore_wait(barrier, 2)

        sends = {}
        for h in range(H):
            sends[(0, h)] = rdma(chunk((my_z - 1) % Z, h), half(r0, h), 0, h, right)
            sends[(0, h)].start()
        tloads = {}
        for h in range(H):
            tloads[h] = pltpu.make_async_copy(
                chunk((my_z - 2) % Z, h), half(t, h), copy_sems.at[h]
            )
            tloads[h].start()

        for h in range(H):
            rdma(chunk(0, h), half(r0, h), 0, h, my_z).wait_recv()
            tloads[h].wait()
            half(r0, h)[:, :] = half(r0, h)[:, :] + half(t, h)[:, :]
            sends[(1, h)] = rdma(half(r0, h), half(r1, h), 1, h, right)
            sends[(1, h)].start()
            tloads[h] = pltpu.make_async_copy(
                chunk((my_z - 3) % Z, h), half(t, h), copy_sems.at[h]
            )
            tloads[h].start()

        for h in range(H):
            rdma(half(r0, h), half(r1, h), 1, h, my_z).wait_recv()
            tloads[h].wait()
            half(r1, h)[:, :] = half(r1, h)[:, :] + half(t, h)[:, :]
            sends[(1, h)].wait_send()
            pl.semaphore_signal(
                credit_sem,
                inc=1,
                device_id=(my_x, my_y, left),
                device_id_type=pl.DeviceIdType.MESH,
            )
            pl.semaphore_wait(credit_sem, 1)
            sends[(2, h)] = rdma(half(r1, h), half(r0, h), 2, h, right)
            sends[(2, h)].start()
            tloads[h] = pltpu.make_async_copy(
                chunk(my_z, h), half(t, h), copy_sems.at[h]
            )
            tloads[h].start()

        outcopies = {}
        for h in range(H):
            rdma(half(r1, h), half(r0, h), 2, h, my_z).wait_recv()
            tloads[h].wait()
            half(r0, h)[:, :] = half(r0, h)[:, :] + half(t, h)[:, :]
            outcopies[h] = pltpu.make_async_copy(
                half(r0, h), half(out_hbm, h), out_sems.at[h]
            )
            outcopies[h].start()

        for h in range(H):
            outcopies[h].wait()
            sends[(0, h)].wait_send()
            sends[(2, h)].wait_send()

    return pl.pallas_call(
        body,
        out_shape=jax.ShapeDtypeStruct((m, n_per), x.dtype),
        in_specs=[pl.BlockSpec(memory_space=pl.ANY)],
        out_specs=pl.BlockSpec(memory_space=pl.ANY),
        scratch_shapes=[
            pltpu.VMEM((m, n_per), x.dtype),
            pltpu.VMEM((m, n_per), x.dtype),
            pltpu.VMEM((m, n_per), x.dtype),
            pltpu.SemaphoreType.DMA((H,)),
            pltpu.SemaphoreType.DMA((H,)),
            pltpu.SemaphoreType.DMA((Z - 1, H)),
            pltpu.SemaphoreType.DMA((Z - 1, H)),
            pltpu.SemaphoreType.REGULAR,
        ],
        compiler_params=pltpu.CompilerParams(
            collective_id=0,
            vmem_limit_bytes=60 * 1024 * 1024,
        ),
    )(x)
